# Initial kernel scaffold; baseline (speedup 1.0000x reference)
import jax
import jax.numpy as jnp
from jax.experimental import pallas as pl


def kernel(x, embedding_table):
    raise NotImplementedError("write your pallas kernel here")


# speedup vs baseline: 1.5977x; 1.5977x over previous
import functools

import jax
import jax.numpy as jnp
from jax import lax
from jax.experimental import pallas as pl
from jax.experimental.pallas import tpu as pltpu
from jax.experimental.pallas import tpu_sc as plsc

_N = 16384
_D = 32
_K = 8192
_RB = 512


def _argmin_body(x_ref, et_ref, idx_ref):
    fx = x_ref[...]
    fxn = fx / jnp.maximum(
        jnp.sqrt(jnp.sum(fx * fx, axis=1, keepdims=True)), 1e-12)
    et = et_ref[...]
    etn = et / jnp.maximum(
        jnp.sqrt(jnp.sum(et * et, axis=0, keepdims=True)), 1e-12)
    dot = jnp.dot(fxn, etn, preferred_element_type=jnp.float32)
    rowsq = jnp.sum(fxn * fxn, axis=1, keepdims=True)
    colsq = jnp.sum(etn * etn, axis=0, keepdims=True)
    d = (rowsq - 2.0 * dot) + colsq
    idx_ref[0, 0, :] = jnp.argmin(d, axis=1).astype(jnp.int32)


_argmin_call = pl.pallas_call(
    _argmin_body,
    grid=(_N // _RB,),
    in_specs=[
        pl.BlockSpec((_RB, _D), lambda i: (i, 0)),
        pl.BlockSpec((_D, _K), lambda i: (0, 0)),
    ],
    out_specs=pl.BlockSpec((1, 1, _RB), lambda i: (i, 0, 0)),
    out_shape=jax.ShapeDtypeStruct((_N // _RB, 1, _RB), jnp.int32),
)


_NC, _NS = 2, 16
_NW = _NC * _NS
_TPW = _N // _NW
_CH = 128
_NCH = _TPW // _CH
_CPW = _K // _NW


@functools.cache
def _make_sc_gather():
    mesh = plsc.VectorSubcoreMesh(
        core_axis_name="c", subcore_axis_name="s",
        num_cores=_NC, num_subcores=_NS)

    @functools.partial(
        pl.kernel,
        mesh=mesh,
        compiler_params=pltpu.CompilerParams(use_tc_tiling_on_sc=False),
        out_type=[
            jax.ShapeDtypeStruct((_N, _D), jnp.float32),
            jax.ShapeDtypeStruct((_K,), jnp.float32),
        ],
        scratch_types=[
            pltpu.VMEM((_CH,), jnp.int32),
            pltpu.VMEM((_CH, _D), jnp.float32),
            pltpu.VMEM((_CH,), jnp.float32),
            pltpu.VMEM((_CPW,), jnp.float32),
            pltpu.VMEM_SHARED((_K,), jnp.float32),
            pltpu.SemaphoreType.DMA,
        ],
    )
    def sc_gather(table_hbm, idx_hbm, out_hbm, counts_hbm,
                  idx_v, rows_v, ones_v, zeros_v, counts_sh, sem):
        wid = lax.axis_index("s") * _NC + lax.axis_index("c")
        for i in range(0, _CPW, 16):
            zeros_v[pl.ds(i, 16)] = jnp.zeros((16,), jnp.float32)
        for i in range(0, _CH, 16):
            ones_v[pl.ds(i, 16)] = jnp.ones((16,), jnp.float32)
        cbase = pl.multiple_of(wid * _CPW, _CPW)
        pltpu.sync_copy(zeros_v, counts_sh.at[pl.ds(cbase, _CPW)])
        plsc.subcore_barrier()
        for j in range(_NCH):
            r = wid * _NCH + j
            pltpu.sync_copy(idx_hbm.at[r], idx_v)
            pltpu.async_copy(table_hbm.at[idx_v], rows_v, sem).wait()
            pltpu.sync_copy(rows_v, out_hbm.at[pl.ds(r * _CH, _CH)])
            pltpu.sync_copy(ones_v, counts_sh.at[idx_v], add=True)
        plsc.subcore_barrier()
        pltpu.sync_copy(counts_sh.at[pl.ds(cbase, _CPW)],
                        counts_hbm.at[pl.ds(cbase, _CPW)])

    return sc_gather


def _loss_body(x_ref, q_ref, c_ref, loss_ref, ent_ref):
    xv = x_ref[...]
    qv = q_ref[...]
    diff = xv - qv
    loss_ref[...] = (jnp.sum(diff * diff) / float(xv.size)).reshape(1, 1)
    c = c_ref[...]
    p = c / jnp.sum(c)
    ent_ref[...] = jnp.sum(p * jnp.log(p + 1e-10)).reshape(1, 1)


_loss_call = pl.pallas_call(
    _loss_body,
    out_shape=[
        jax.ShapeDtypeStruct((1, 1), jnp.float32),
        jax.ShapeDtypeStruct((1, 1), jnp.float32),
    ],
)


def kernel(x, embedding_table):
    B, C, H, W = x.shape
    D, K = embedding_table.shape
    flat_x = jnp.transpose(x, (0, 2, 3, 1)).reshape(-1, D)
    idx3 = _argmin_call(flat_x, embedding_table)
    encoding_indices = idx3.reshape(-1)
    idx2d = idx3.reshape(_N // _CH, _CH)
    table_t = embedding_table.T
    quant, counts = _make_sc_gather()(table_t, idx2d)
    loss2, ent2 = _loss_call(flat_x, quant, counts.reshape(64, 128))
    loss = loss2.reshape(())
    ent = ent2.reshape(())
    qx = jnp.transpose(quant.reshape(B, H, W, D), (0, 3, 1, 2))
    return (qx, loss, loss, ent, encoding_indices.reshape(B, -1))
